# baseline (speedup 1.0000x reference)
import functools

import jax
import jax.numpy as jnp
from jax import lax
from jax.experimental import pallas as pl
from jax.experimental.pallas import tpu as pltpu
from jax.experimental.pallas import tpu_sc as plsc

_NC = 2
_NS = 16
_NW = _NC * _NS
_L = 16


@functools.lru_cache(maxsize=None)
def _build(B, K, D):
    assert B % _NW == 0 and D % _L == 0
    BPW = B // _NW
    G = max(1, 128 // K)
    assert BPW % G == 0
    IDX = G * K
    NG = BPW // G
    NCH = D // _L
    RU = 16
    NBUF = 4
    assert K % RU == 0 and NG % NBUF == 0

    mesh = plsc.VectorSubcoreMesh(core_axis_name="c", subcore_axis_name="s")

    @functools.partial(
        pl.kernel,
        out_type=jax.ShapeDtypeStruct((B, D), jnp.float32),
        mesh=mesh,
        scratch_types=[
            pltpu.VMEM((NG, IDX), jnp.int32),
            [pltpu.VMEM((IDX, D), jnp.float32) for _ in range(NBUF)],
            pltpu.VMEM((BPW, D), jnp.float32),
            [pltpu.SemaphoreType.DMA for _ in range(NBUF)],
        ],
    )
    def body(feats_hbm, nb_hbm, out_hbm, idx_v, rows, out_v, sems):
        wid = lax.axis_index("s") * _NC + lax.axis_index("c")
        pltpu.sync_copy(nb_hbm.at[wid], idx_v)

        def accum(rows_v, g):
            for b in range(G):
                def rbody(r, accs):
                    for u in range(RU):
                        row = b * K + r * RU + u
                        accs = tuple(
                            accs[c] + rows_v[row, pl.ds(c * _L, _L)]
                            for c in range(NCH)
                        )
                    return accs
                accs = lax.fori_loop(
                    0, K // RU, rbody,
                    tuple(jnp.zeros((_L,), jnp.float32) for _ in range(NCH)),
                )
                for c in range(NCH):
                    out_v[g * G + b, pl.ds(c * _L, _L)] = accs[c] * (1.0 / K)

        for p in range(NBUF - 1):
            pltpu.async_copy(feats_hbm.at[idx_v.at[p]], rows[p], sems[p])

        def ring(q, carry):
            g0 = NBUF * q
            for p in range(NBUF):
                g = g0 + p
                nxt = g + NBUF - 1

                @pl.when(nxt < NG)
                def _():
                    pltpu.async_copy(
                        feats_hbm.at[idx_v.at[nxt]],
                        rows[(p + NBUF - 1) % NBUF],
                        sems[(p + NBUF - 1) % NBUF],
                    )

                pltpu.make_async_copy(
                    feats_hbm.at[idx_v.at[g]], rows[p], sems[p]).wait()
                accum(rows[p], g)
            return carry

        lax.fori_loop(0, NG // NBUF, ring, 0)
        pltpu.sync_copy(out_v, out_hbm.at[pl.ds(wid * BPW, BPW)])

    return body


def kernel(feats, node_batches):
    B, K = node_batches.shape
    V, D = feats.shape
    nb = node_batches.reshape(-1).astype(jnp.int32)
    G = max(1, 128 // K)
    nb = nb.reshape(_NW, (B // _NW) // G, G * K)
    return _build(B, K, D)(feats, nb)


# speedup vs baseline: 1.3832x; 1.3832x over previous
import functools

import jax
import jax.numpy as jnp
from jax import lax
from jax.experimental import pallas as pl
from jax.experimental.pallas import tpu as pltpu
from jax.experimental.pallas import tpu_sc as plsc

_NC = 2
_NS = 16
_NW = _NC * _NS
_L = 16


@functools.lru_cache(maxsize=None)
def _build(B, K, D):
    assert B % _NW == 0 and D % _L == 0
    BPW = B // _NW
    G = 1
    assert BPW % G == 0
    IDX = G * K
    NG = BPW // G
    NCH = D // _L
    RU = 8
    NBUF = 8
    assert K % RU == 0 and NG % NBUF == 0

    mesh = plsc.VectorSubcoreMesh(core_axis_name="c", subcore_axis_name="s")

    @functools.partial(
        pl.kernel,
        out_type=jax.ShapeDtypeStruct((B, D), jnp.float32),
        mesh=mesh,
        scratch_types=[
            pltpu.VMEM((NG, IDX), jnp.int32),
            [pltpu.VMEM((IDX, D), jnp.float32) for _ in range(NBUF)],
            pltpu.VMEM((BPW, D), jnp.float32),
            [pltpu.SemaphoreType.DMA for _ in range(NBUF)],
        ],
    )
    def body(feats_hbm, nb_hbm, out_hbm, idx_v, rows, out_v, sems):
        wid = lax.axis_index("s") * _NC + lax.axis_index("c")
        pltpu.sync_copy(nb_hbm.at[wid], idx_v)

        def accum(rows_v, g):
            for b in range(G):
                def rbody(r, accs):
                    for u in range(RU):
                        row = b * K + r * RU + u
                        accs = tuple(
                            accs[c] + rows_v[row, pl.ds(c * _L, _L)]
                            for c in range(NCH)
                        )
                    return accs
                accs = lax.fori_loop(
                    0, K // RU, rbody,
                    tuple(jnp.zeros((_L,), jnp.float32) for _ in range(NCH)),
                )
                for c in range(NCH):
                    out_v[g * G + b, pl.ds(c * _L, _L)] = accs[c] * (1.0 / K)

        for p in range(NBUF - 1):
            pltpu.async_copy(feats_hbm.at[idx_v.at[p]], rows[p], sems[p])

        def ring(q, carry):
            g0 = NBUF * q
            for p in range(NBUF):
                g = g0 + p
                nxt = g + NBUF - 1

                @pl.when(nxt < NG)
                def _():
                    pltpu.async_copy(
                        feats_hbm.at[idx_v.at[nxt]],
                        rows[(p + NBUF - 1) % NBUF],
                        sems[(p + NBUF - 1) % NBUF],
                    )

                pltpu.make_async_copy(
                    feats_hbm.at[idx_v.at[g]], rows[p], sems[p]).wait()
                accum(rows[p], g)
            return carry

        lax.fori_loop(0, NG // NBUF, ring, 0)
        pltpu.sync_copy(out_v, out_hbm.at[pl.ds(wid * BPW, BPW)])

    return body


def kernel(feats, node_batches):
    B, K = node_batches.shape
    V, D = feats.shape
    nb = node_batches.reshape(-1).astype(jnp.int32)
    G = 1
    nb = nb.reshape(_NW, (B // _NW) // G, G * K)
    return _build(B, K, D)(feats, nb)
